# Initial kernel scaffold; baseline (speedup 1.0000x reference)
import jax
import jax.numpy as jnp
from jax.experimental import pallas as pl


def kernel(x, edge_index, W_l0, b_l0, W_r0, gamma0, beta0, W_l1, b_l1, W_r1, gamma1, beta1, skip_W, skip_b):
    raise NotImplementedError("write your pallas kernel here")


# speedup vs baseline: 3.2316x; 3.2316x over previous
import dataclasses
import functools

import jax
import jax.numpy as jnp
from jax import lax
from jax.experimental import pallas as pl
from jax.experimental.pallas import tpu as pltpu
from jax.experimental.pallas import tpu_sc as plsc

N = 10000
E = 320000
H = 128

NCORES = 2
NSUB = 16
NWORK = NCORES * NSUB
CHUNK = 128
CHUNKS_PER_WORKER = 80
TOTAL_CHUNKS = NWORK * CHUNKS_PER_WORKER
E_PAD = TOTAL_CHUNKS * CHUNK
IDX_GRP = 8
ROWS_PER_SUB = 640
N_PAD = NSUB * ROWS_PER_SUB

ROW_BLK = 1000
NBLK = N // ROW_BLK


def _sc_aggregate(h, src_p, dst_p, with_cnt):
    del with_cnt
    mesh = plsc.VectorSubcoreMesh(core_axis_name="c", subcore_axis_name="s")

    @functools.partial(
        pl.kernel,
        out_type=[jax.ShapeDtypeStruct((NCORES * N_PAD, H), jnp.float32)],
        mesh=mesh,
        scratch_types=[
            pltpu.VMEM((IDX_GRP, CHUNK), jnp.int32),
            pltpu.VMEM((IDX_GRP, CHUNK), jnp.int32),
            pltpu.VMEM((CHUNK, H), jnp.float32),
            pltpu.VMEM_SHARED((N_PAD, H), jnp.float32),
            pltpu.SemaphoreType.DMA,
        ])
    def body(h_hbm, src_hbm, dst_hbm, psum_hbm, srcv, dstv, rows, acc, sem):
        c = lax.axis_index("c")
        s = lax.axis_index("s")
        w = c * NSUB + s

        zv = jnp.zeros((16,), jnp.float32)

        @pl.loop(0, CHUNK)
        def _(r):
            @pl.loop(0, H // 16)
            def _(k):
                rows[r, pl.ds(k * 16, 16)] = zv

        @pl.loop(0, ROWS_PER_SUB // CHUNK)
        def _(j):
            pltpu.sync_copy(
                rows, acc.at[pl.ds(s * ROWS_PER_SUB + j * CHUNK, CHUNK)])

        plsc.subcore_barrier()

        wbase = w * CHUNKS_PER_WORKER

        @pl.loop(0, CHUNKS_PER_WORKER // IDX_GRP)
        def _(g):
            pltpu.sync_copy(src_hbm.at[pl.ds(wbase + g * IDX_GRP, IDX_GRP)],
                            srcv)
            pltpu.sync_copy(dst_hbm.at[pl.ds(wbase + g * IDX_GRP, IDX_GRP)],
                            dstv)

            @pl.loop(0, IDX_GRP)
            def _(j):
                pltpu.async_copy(h_hbm.at[srcv.at[j]], rows, sem).wait()
                pltpu.sync_copy(rows, acc.at[dstv.at[j]], add=True)

        plsc.subcore_barrier()

        pltpu.sync_copy(acc.at[pl.ds(s * ROWS_PER_SUB, ROWS_PER_SUB)],
                        psum_hbm.at[pl.ds(c * N_PAD + s * ROWS_PER_SUB,
                                          ROWS_PER_SUB)])

    return body(h, src_p, dst_p)


def _sc_counts(dst_p):
    mesh = plsc.VectorSubcoreMesh(core_axis_name="c", subcore_axis_name="s")
    cp = pltpu.CompilerParams()
    if "needs_layout_passes" in pltpu.CompilerParams.__dataclass_fields__:
        cp = dataclasses.replace(cp, needs_layout_passes=False)

    @functools.partial(
        pl.kernel,
        out_type=[jax.ShapeDtypeStruct((NWORK, N_PAD), jnp.float32)],
        mesh=mesh,
        compiler_params=cp,
        scratch_types=[
            pltpu.VMEM((IDX_GRP, CHUNK), jnp.int32),
            pltpu.VMEM((N_PAD,), jnp.float32),
        ])
    def body(dst_hbm, cnt_hbm, dstv, hist):
        c = lax.axis_index("c")
        s = lax.axis_index("s")
        w = c * NSUB + s
        zv = jnp.zeros((16,), jnp.float32)
        ov = jnp.ones((16,), jnp.float32)

        @pl.loop(0, N_PAD // 16)
        def _(i):
            hist[pl.ds(i * 16, 16)] = zv

        wbase = w * CHUNKS_PER_WORKER

        @pl.loop(0, CHUNKS_PER_WORKER // IDX_GRP)
        def _(g):
            pltpu.sync_copy(dst_hbm.at[pl.ds(wbase + g * IDX_GRP, IDX_GRP)],
                            dstv)

            @pl.loop(0, IDX_GRP)
            def _(j):
                @pl.loop(0, CHUNK // 16)
                def _(k):
                    dv = dstv[j, pl.ds(k * 16, 16)]
                    plsc.addupdate_scatter(hist, [dv], ov)

        pltpu.sync_copy(hist, cnt_hbm.at[w])

    return body(dst_p)


def _tc_pre(h, W_r, skip_W, skip_b2):
    def body(h_ref, wr_ref, sw_ref, sb_ref, z_ref, res_ref):
        hb = h_ref[...]
        z_ref[...] = jnp.dot(hb, wr_ref[...],
                             preferred_element_type=jnp.float32)
        res_ref[...] = jnp.dot(hb, sw_ref[...],
                               preferred_element_type=jnp.float32) + sb_ref[...]

    return pl.pallas_call(
        body,
        grid=(NBLK,),
        in_specs=[
            pl.BlockSpec((ROW_BLK, H), lambda i: (i, 0)),
            pl.BlockSpec((H, H), lambda i: (0, 0)),
            pl.BlockSpec((H, H), lambda i: (0, 0)),
            pl.BlockSpec((1, H), lambda i: (0, 0)),
        ],
        out_specs=[
            pl.BlockSpec((ROW_BLK, H), lambda i: (i, 0)),
            pl.BlockSpec((ROW_BLK, H), lambda i: (i, 0)),
        ],
        out_shape=[jax.ShapeDtypeStruct((N, H), jnp.float32)] * 2,
    )(h, W_r, skip_W, skip_b2)


def _tc_post_a(psum, pcnt, z, W_l, b_l2):
    def body(ps_ref, pc_ref, z_ref, wl_ref, bl_ref, pre_ref, st_ref, acc_ref):
        i = pl.program_id(0)
        ssum = ps_ref[0] + ps_ref[1]
        cnt = jnp.sum(pc_ref[...], axis=1, keepdims=True)
        agg = ssum / jnp.maximum(cnt, 1.0)
        out = (jnp.dot(agg, wl_ref[...], preferred_element_type=jnp.float32)
               + bl_ref[...] + z_ref[...])
        nrm2 = jnp.sum(out * out, axis=1, keepdims=True)
        out = out * lax.rsqrt(jnp.maximum(nrm2, 1e-24))
        pre_ref[...] = out

        @pl.when(i == 0)
        def _():
            acc_ref[...] = jnp.zeros((8, H), jnp.float32)

        acc_ref[0:1] += jnp.sum(out, axis=0, keepdims=True)
        acc_ref[1:2] += jnp.sum(out * out, axis=0, keepdims=True)

        @pl.when(i == NBLK - 1)
        def _():
            st_ref[...] = acc_ref[...]

    return pl.pallas_call(
        body,
        grid=(NBLK,),
        in_specs=[
            pl.BlockSpec((NCORES, ROW_BLK, H), lambda i: (0, i, 0)),
            pl.BlockSpec((ROW_BLK, NWORK), lambda i: (i, 0)),
            pl.BlockSpec((ROW_BLK, H), lambda i: (i, 0)),
            pl.BlockSpec((H, H), lambda i: (0, 0)),
            pl.BlockSpec((1, H), lambda i: (0, 0)),
        ],
        out_specs=[
            pl.BlockSpec((ROW_BLK, H), lambda i: (i, 0)),
            pl.BlockSpec((8, H), lambda i: (0, 0)),
        ],
        out_shape=[
            jax.ShapeDtypeStruct((N, H), jnp.float32),
            jax.ShapeDtypeStruct((8, H), jnp.float32),
        ],
        scratch_shapes=[pltpu.VMEM((8, H), jnp.float32)],
    )(psum, pcnt, z, W_l, b_l2)


def _tc_post_b(pre, res, stats, gamma2, beta2):
    def body(pre_ref, res_ref, st_ref, g_ref, b_ref, out_ref):
        mu = st_ref[0:1] / N
        var = st_ref[1:2] / N - mu * mu
        scale = g_ref[...] * lax.rsqrt(var + 1e-5)
        y = (pre_ref[...] - mu) * scale + b_ref[...]
        out_ref[...] = jnp.where(y >= 0, y, 0.1 * y) + res_ref[...]

    return pl.pallas_call(
        body,
        grid=(NBLK,),
        in_specs=[
            pl.BlockSpec((ROW_BLK, H), lambda i: (i, 0)),
            pl.BlockSpec((ROW_BLK, H), lambda i: (i, 0)),
            pl.BlockSpec((8, H), lambda i: (0, 0)),
            pl.BlockSpec((1, H), lambda i: (0, 0)),
            pl.BlockSpec((1, H), lambda i: (0, 0)),
        ],
        out_specs=pl.BlockSpec((ROW_BLK, H), lambda i: (i, 0)),
        out_shape=jax.ShapeDtypeStruct((N, H), jnp.float32),
    )(pre, res, stats, gamma2, beta2)


def kernel(x, edge_index, W_l0, b_l0, W_r0, gamma0, beta0,
           W_l1, b_l1, W_r1, gamma1, beta1, skip_W, skip_b):
    src = edge_index[0]
    dst = edge_index[1]
    pad = E_PAD - E
    src_p = jnp.concatenate([src, jnp.zeros((pad,), jnp.int32)])
    src_p = src_p.reshape(TOTAL_CHUNKS, CHUNK)
    junk = N + (lax.iota(jnp.int32, pad) % (N_PAD - N))
    dst_p = jnp.concatenate([dst, junk]).reshape(TOTAL_CHUNKS, CHUNK)

    b_l0r = b_l0.reshape(1, H)
    b_l1r = b_l1.reshape(1, H)
    sbr = skip_b.reshape(1, H)
    g0r = gamma0.reshape(1, H)
    g1r = gamma1.reshape(1, H)
    be0r = beta0.reshape(1, H)
    be1r = beta1.reshape(1, H)

    z1, res1 = _tc_pre(x, W_r0, skip_W, sbr)
    (pc1,) = _sc_counts(dst_p)
    (ps1,) = _sc_aggregate(x, src_p, dst_p, with_cnt=True)
    ps1 = ps1.reshape(NCORES, N_PAD, H)
    pc1 = pc1.T
    pre1, st1 = _tc_post_a(ps1, pc1, z1, W_l0, b_l0r)
    h1 = _tc_post_b(pre1, res1, st1, g0r, be0r)

    z2, res2 = _tc_pre(h1, W_r1, skip_W, sbr)
    (ps2,) = _sc_aggregate(h1, src_p, dst_p, with_cnt=False)
    ps2 = ps2.reshape(NCORES, N_PAD, H)
    pre2, st2 = _tc_post_a(ps2, pc1, z2, W_l1, b_l1r)
    out = _tc_post_b(pre2, res2, st2, g1r, be1r)
    return out


# speedup vs baseline: 8.2826x; 2.5630x over previous
import dataclasses
import functools

import jax
import jax.numpy as jnp
from jax import lax
from jax.experimental import pallas as pl
from jax.experimental.pallas import tpu as pltpu
from jax.experimental.pallas import tpu_sc as plsc

N = 10000
E = 320000
H = 128

NCORES = 2
NSUB = 16
NWORK = NCORES * NSUB
CHUNK = 128
CHUNKS_PER_WORKER = 80
TOTAL_CHUNKS = NWORK * CHUNKS_PER_WORKER
E_PAD = TOTAL_CHUNKS * CHUNK
IDX_GRP = 8
ROWS_PER_SUB = 640
N_PAD = NSUB * ROWS_PER_SUB

ROW_BLK = 1000
NBLK = N // ROW_BLK


def _sc_aggregate(h, src_p, dst_p, with_cnt):
    del with_cnt
    mesh = plsc.VectorSubcoreMesh(core_axis_name="c", subcore_axis_name="s")

    @functools.partial(
        pl.kernel,
        out_type=[jax.ShapeDtypeStruct((NCORES * N_PAD, H), jnp.float32)],
        mesh=mesh,
        scratch_types=[
            pltpu.VMEM((IDX_GRP, CHUNK), jnp.int32),
            pltpu.VMEM((IDX_GRP, CHUNK), jnp.int32),
            pltpu.VMEM((CHUNK, H), jnp.float32),
            pltpu.VMEM_SHARED((N_PAD, H), jnp.float32),
            pltpu.SemaphoreType.DMA,
        ])
    def body(h_hbm, src_hbm, dst_hbm, psum_hbm, srcv, dstv, rows, acc, sem):
        c = lax.axis_index("c")
        s = lax.axis_index("s")
        w = c * NSUB + s

        zv = jnp.zeros((16,), jnp.float32)

        @pl.loop(0, CHUNK)
        def _(r):
            @pl.loop(0, H // 16)
            def _(k):
                rows[r, pl.ds(k * 16, 16)] = zv

        @pl.loop(0, ROWS_PER_SUB // CHUNK)
        def _(j):
            pltpu.sync_copy(
                rows, acc.at[pl.ds(s * ROWS_PER_SUB + j * CHUNK, CHUNK)])

        plsc.subcore_barrier()

        wbase = w * CHUNKS_PER_WORKER

        @pl.loop(0, CHUNKS_PER_WORKER // IDX_GRP)
        def _(g):
            pltpu.sync_copy(src_hbm.at[pl.ds(wbase + g * IDX_GRP, IDX_GRP)],
                            srcv)
            pltpu.sync_copy(dst_hbm.at[pl.ds(wbase + g * IDX_GRP, IDX_GRP)],
                            dstv)

            @pl.loop(0, IDX_GRP)
            def _(j):
                pltpu.async_copy(h_hbm.at[srcv.at[j]], rows, sem).wait()
                pltpu.sync_copy(rows, acc.at[dstv.at[j]], add=True)

        plsc.subcore_barrier()

        pltpu.sync_copy(acc.at[pl.ds(s * ROWS_PER_SUB, ROWS_PER_SUB)],
                        psum_hbm.at[pl.ds(c * N_PAD + s * ROWS_PER_SUB,
                                          ROWS_PER_SUB)])

    return body(h, src_p, dst_p)


def _sc_counts(dst_p):
    mesh = plsc.VectorSubcoreMesh(core_axis_name="c", subcore_axis_name="s")
    cp = pltpu.CompilerParams()
    if "needs_layout_passes" in pltpu.CompilerParams.__dataclass_fields__:
        cp = dataclasses.replace(cp, needs_layout_passes=False)

    @functools.partial(
        pl.kernel,
        out_type=[jax.ShapeDtypeStruct((NWORK, N_PAD), jnp.float32)],
        mesh=mesh,
        compiler_params=cp,
        scratch_types=[
            pltpu.VMEM((IDX_GRP, CHUNK), jnp.int32),
            pltpu.VMEM((N_PAD,), jnp.float32),
        ])
    def body(dst_hbm, cnt_hbm, dstv, hist):
        c = lax.axis_index("c")
        s = lax.axis_index("s")
        w = c * NSUB + s
        zv = jnp.zeros((16,), jnp.float32)
        ov = jnp.ones((16,), jnp.float32)

        @pl.loop(0, N_PAD // 16)
        def _(i):
            hist[pl.ds(i * 16, 16)] = zv

        wbase = w * CHUNKS_PER_WORKER

        @pl.loop(0, CHUNKS_PER_WORKER // IDX_GRP)
        def _(g):
            pltpu.sync_copy(dst_hbm.at[pl.ds(wbase + g * IDX_GRP, IDX_GRP)],
                            dstv)

            @pl.loop(0, IDX_GRP)
            def _(j):
                @pl.loop(0, CHUNK // 16)
                def _(k):
                    dv = dstv[j, pl.ds(k * 16, 16)]
                    plsc.addupdate_scatter(hist, [dv], ov)

        pltpu.sync_copy(hist, cnt_hbm.at[w])

    return body(dst_p)


def _tc_pre(h, W_r, skip_W, skip_b2):
    def body(h_ref, wr_ref, sw_ref, sb_ref, z_ref, res_ref):
        hb = h_ref[...]
        z_ref[...] = jnp.dot(hb, wr_ref[...],
                             preferred_element_type=jnp.float32)
        res_ref[...] = jnp.dot(hb, sw_ref[...],
                               preferred_element_type=jnp.float32) + sb_ref[...]

    return pl.pallas_call(
        body,
        grid=(NBLK,),
        in_specs=[
            pl.BlockSpec((ROW_BLK, H), lambda i: (i, 0)),
            pl.BlockSpec((H, H), lambda i: (0, 0)),
            pl.BlockSpec((H, H), lambda i: (0, 0)),
            pl.BlockSpec((1, H), lambda i: (0, 0)),
        ],
        out_specs=[
            pl.BlockSpec((ROW_BLK, H), lambda i: (i, 0)),
            pl.BlockSpec((ROW_BLK, H), lambda i: (i, 0)),
        ],
        out_shape=[jax.ShapeDtypeStruct((N, H), jnp.float32)] * 2,
    )(h, W_r, skip_W, skip_b2)


def _tc_post_a(psum, pcnt, z, W_l, b_l2):
    def body(ps_ref, pc_ref, z_ref, wl_ref, bl_ref, pre_ref, st_ref, acc_ref):
        i = pl.program_id(0)
        ssum = ps_ref[0] + ps_ref[1]
        cnt = jnp.sum(pc_ref[...], axis=1, keepdims=True)
        agg = ssum / jnp.maximum(cnt, 1.0)
        out = (jnp.dot(agg, wl_ref[...], preferred_element_type=jnp.float32)
               + bl_ref[...] + z_ref[...])
        nrm2 = jnp.sum(out * out, axis=1, keepdims=True)
        out = out * lax.rsqrt(jnp.maximum(nrm2, 1e-24))
        pre_ref[...] = out

        @pl.when(i == 0)
        def _():
            acc_ref[...] = jnp.zeros((8, H), jnp.float32)

        acc_ref[0:1] += jnp.sum(out, axis=0, keepdims=True)
        acc_ref[1:2] += jnp.sum(out * out, axis=0, keepdims=True)

        @pl.when(i == NBLK - 1)
        def _():
            st_ref[...] = acc_ref[...]

    return pl.pallas_call(
        body,
        grid=(NBLK,),
        in_specs=[
            pl.BlockSpec((NCORES, ROW_BLK, H), lambda i: (0, i, 0)),
            pl.BlockSpec((ROW_BLK, NWORK), lambda i: (i, 0)),
            pl.BlockSpec((ROW_BLK, H), lambda i: (i, 0)),
            pl.BlockSpec((H, H), lambda i: (0, 0)),
            pl.BlockSpec((1, H), lambda i: (0, 0)),
        ],
        out_specs=[
            pl.BlockSpec((ROW_BLK, H), lambda i: (i, 0)),
            pl.BlockSpec((8, H), lambda i: (0, 0)),
        ],
        out_shape=[
            jax.ShapeDtypeStruct((N, H), jnp.float32),
            jax.ShapeDtypeStruct((8, H), jnp.float32),
        ],
        scratch_shapes=[pltpu.VMEM((8, H), jnp.float32)],
    )(psum, pcnt, z, W_l, b_l2)


def _tc_post_b(pre, res, stats, gamma2, beta2):
    def body(pre_ref, res_ref, st_ref, g_ref, b_ref, out_ref):
        mu = st_ref[0:1] / N
        var = st_ref[1:2] / N - mu * mu
        scale = g_ref[...] * lax.rsqrt(var + 1e-5)
        y = (pre_ref[...] - mu) * scale + b_ref[...]
        out_ref[...] = jnp.where(y >= 0, y, 0.1 * y) + res_ref[...]

    return pl.pallas_call(
        body,
        grid=(NBLK,),
        in_specs=[
            pl.BlockSpec((ROW_BLK, H), lambda i: (i, 0)),
            pl.BlockSpec((ROW_BLK, H), lambda i: (i, 0)),
            pl.BlockSpec((8, H), lambda i: (0, 0)),
            pl.BlockSpec((1, H), lambda i: (0, 0)),
            pl.BlockSpec((1, H), lambda i: (0, 0)),
        ],
        out_specs=pl.BlockSpec((ROW_BLK, H), lambda i: (i, 0)),
        out_shape=jax.ShapeDtypeStruct((N, H), jnp.float32),
    )(pre, res, stats, gamma2, beta2)


def kernel(x, edge_index, W_l0, b_l0, W_r0, gamma0, beta0,
           W_l1, b_l1, W_r1, gamma1, beta1, skip_W, skip_b):
    src = edge_index[0]
    dst = edge_index[1]
    pad = E_PAD - E
    pad_iota = lax.iota(jnp.int32, pad)
    src_p = jnp.concatenate([src, (pad_iota * 37) % N])
    src_p = src_p.reshape(TOTAL_CHUNKS, CHUNK)
    junk = N + (pad_iota % (N_PAD - N))
    dst_p = jnp.concatenate([dst, junk]).reshape(TOTAL_CHUNKS, CHUNK)

    b_l0r = b_l0.reshape(1, H)
    b_l1r = b_l1.reshape(1, H)
    sbr = skip_b.reshape(1, H)
    g0r = gamma0.reshape(1, H)
    g1r = gamma1.reshape(1, H)
    be0r = beta0.reshape(1, H)
    be1r = beta1.reshape(1, H)

    z1, res1 = _tc_pre(x, W_r0, skip_W, sbr)
    (pc1,) = _sc_counts(dst_p)
    (ps1,) = _sc_aggregate(x, src_p, dst_p, with_cnt=True)
    ps1 = ps1.reshape(NCORES, N_PAD, H)
    pc1 = pc1.T
    pre1, st1 = _tc_post_a(ps1, pc1, z1, W_l0, b_l0r)
    h1 = _tc_post_b(pre1, res1, st1, g0r, be0r)

    z2, res2 = _tc_pre(h1, W_r1, skip_W, sbr)
    (ps2,) = _sc_aggregate(h1, src_p, dst_p, with_cnt=False)
    ps2 = ps2.reshape(NCORES, N_PAD, H)
    pre2, st2 = _tc_post_a(ps2, pc1, z2, W_l1, b_l1r)
    out = _tc_post_b(pre2, res2, st2, g1r, be1r)
    return out


# speedup vs baseline: 11.6699x; 1.4090x over previous
import dataclasses
import functools

import jax
import jax.numpy as jnp
from jax import lax
from jax.experimental import pallas as pl
from jax.experimental.pallas import tpu as pltpu
from jax.experimental.pallas import tpu_sc as plsc

N = 10000
E = 320000
H = 128

NCORES = 2
NSUB = 16
NWORK = NCORES * NSUB
CHUNK = 128
CHUNKS_PER_WORKER = 80
TOTAL_CHUNKS = NWORK * CHUNKS_PER_WORKER
E_PAD = TOTAL_CHUNKS * CHUNK
IDX_GRP = 16
ROWS_PER_SUB = 640
N_PAD = NSUB * ROWS_PER_SUB

ROW_BLK = 1000
NBLK = N // ROW_BLK


def _sc_aggregate(h, src_p, dst_p, with_cnt):
    del with_cnt
    mesh = plsc.VectorSubcoreMesh(core_axis_name="c", subcore_axis_name="s")

    @functools.partial(
        pl.kernel,
        out_type=[jax.ShapeDtypeStruct((NCORES * N_PAD, H), jnp.float32)],
        mesh=mesh,
        scratch_types=[
            pltpu.VMEM((IDX_GRP, CHUNK), jnp.int32),
            pltpu.VMEM((IDX_GRP, CHUNK), jnp.int32),
            pltpu.VMEM((CHUNK, H), jnp.float32),
            pltpu.VMEM((CHUNK, H), jnp.float32),
            pltpu.VMEM_SHARED((N_PAD, H), jnp.float32),
            pltpu.SemaphoreType.DMA,
            pltpu.SemaphoreType.DMA,
        ])
    def body(h_hbm, src_hbm, dst_hbm, psum_hbm, srcv, dstv, rows0, rows1,
             acc, sem0, sem1):
        c = lax.axis_index("c")
        s = lax.axis_index("s")
        w = c * NSUB + s
        rowsb = (rows0, rows1)
        sems = (sem0, sem1)

        zv = jnp.zeros((16,), jnp.float32)

        @pl.loop(0, CHUNK)
        def _(r):
            @pl.loop(0, H // 16)
            def _(k):
                rows0[r, pl.ds(k * 16, 16)] = zv

        @pl.loop(0, ROWS_PER_SUB // CHUNK)
        def _(j):
            pltpu.sync_copy(
                rows0, acc.at[pl.ds(s * ROWS_PER_SUB + j * CHUNK, CHUNK)])

        plsc.subcore_barrier()

        wbase = w * CHUNKS_PER_WORKER

        @pl.loop(0, CHUNKS_PER_WORKER // IDX_GRP)
        def _(g):
            pltpu.sync_copy(src_hbm.at[pl.ds(wbase + g * IDX_GRP, IDX_GRP)],
                            srcv)
            pltpu.sync_copy(dst_hbm.at[pl.ds(wbase + g * IDX_GRP, IDX_GRP)],
                            dstv)

            handle = pltpu.async_copy(h_hbm.at[srcv.at[0]], rows0, sem0)
            for j in range(IDX_GRP):
                b = j & 1
                nxt = None
                if j + 1 < IDX_GRP:
                    nxt = pltpu.async_copy(h_hbm.at[srcv.at[j + 1]],
                                           rowsb[1 - b], sems[1 - b])
                handle.wait()
                pltpu.sync_copy(rowsb[b], acc.at[dstv.at[j]], add=True)
                handle = nxt

        plsc.subcore_barrier()

        pltpu.sync_copy(acc.at[pl.ds(s * ROWS_PER_SUB, ROWS_PER_SUB)],
                        psum_hbm.at[pl.ds(c * N_PAD + s * ROWS_PER_SUB,
                                          ROWS_PER_SUB)])

    return body(h, src_p, dst_p)


def _sc_counts(dst_p):
    mesh = plsc.VectorSubcoreMesh(core_axis_name="c", subcore_axis_name="s")
    cp = pltpu.CompilerParams()
    if "needs_layout_passes" in pltpu.CompilerParams.__dataclass_fields__:
        cp = dataclasses.replace(cp, needs_layout_passes=False)

    @functools.partial(
        pl.kernel,
        out_type=[jax.ShapeDtypeStruct((NWORK, N_PAD), jnp.float32)],
        mesh=mesh,
        compiler_params=cp,
        scratch_types=[
            pltpu.VMEM((IDX_GRP, CHUNK), jnp.int32),
            pltpu.VMEM((N_PAD,), jnp.float32),
        ])
    def body(dst_hbm, cnt_hbm, dstv, hist):
        c = lax.axis_index("c")
        s = lax.axis_index("s")
        w = c * NSUB + s
        zv = jnp.zeros((16,), jnp.float32)
        ov = jnp.ones((16,), jnp.float32)

        @pl.loop(0, N_PAD // 16)
        def _(i):
            hist[pl.ds(i * 16, 16)] = zv

        wbase = w * CHUNKS_PER_WORKER

        @pl.loop(0, CHUNKS_PER_WORKER // IDX_GRP)
        def _(g):
            pltpu.sync_copy(dst_hbm.at[pl.ds(wbase + g * IDX_GRP, IDX_GRP)],
                            dstv)

            @pl.loop(0, IDX_GRP)
            def _(j):
                @pl.loop(0, CHUNK // 16)
                def _(k):
                    dv = dstv[j, pl.ds(k * 16, 16)]
                    plsc.addupdate_scatter(hist, [dv], ov)

        pltpu.sync_copy(hist, cnt_hbm.at[w])

    return body(dst_p)


def _tc_pre(h, W_r, skip_W, skip_b2):
    def body(h_ref, wr_ref, sw_ref, sb_ref, z_ref, res_ref):
        hb = h_ref[...]
        z_ref[...] = jnp.dot(hb, wr_ref[...],
                             preferred_element_type=jnp.float32)
        res_ref[...] = jnp.dot(hb, sw_ref[...],
                               preferred_element_type=jnp.float32) + sb_ref[...]

    return pl.pallas_call(
        body,
        grid=(NBLK,),
        in_specs=[
            pl.BlockSpec((ROW_BLK, H), lambda i: (i, 0)),
            pl.BlockSpec((H, H), lambda i: (0, 0)),
            pl.BlockSpec((H, H), lambda i: (0, 0)),
            pl.BlockSpec((1, H), lambda i: (0, 0)),
        ],
        out_specs=[
            pl.BlockSpec((ROW_BLK, H), lambda i: (i, 0)),
            pl.BlockSpec((ROW_BLK, H), lambda i: (i, 0)),
        ],
        out_shape=[jax.ShapeDtypeStruct((N, H), jnp.float32)] * 2,
    )(h, W_r, skip_W, skip_b2)


def _tc_post_a(psum, pcnt, z, W_l, b_l2):
    def body(ps_ref, pc_ref, z_ref, wl_ref, bl_ref, pre_ref, st_ref, acc_ref):
        i = pl.program_id(0)
        ssum = ps_ref[0] + ps_ref[1]
        cnt = jnp.sum(pc_ref[...], axis=1, keepdims=True)
        agg = ssum / jnp.maximum(cnt, 1.0)
        out = (jnp.dot(agg, wl_ref[...], preferred_element_type=jnp.float32)
               + bl_ref[...] + z_ref[...])
        nrm2 = jnp.sum(out * out, axis=1, keepdims=True)
        out = out * lax.rsqrt(jnp.maximum(nrm2, 1e-24))
        pre_ref[...] = out

        @pl.when(i == 0)
        def _():
            acc_ref[...] = jnp.zeros((8, H), jnp.float32)

        acc_ref[0:1] += jnp.sum(out, axis=0, keepdims=True)
        acc_ref[1:2] += jnp.sum(out * out, axis=0, keepdims=True)

        @pl.when(i == NBLK - 1)
        def _():
            st_ref[...] = acc_ref[...]

    return pl.pallas_call(
        body,
        grid=(NBLK,),
        in_specs=[
            pl.BlockSpec((NCORES, ROW_BLK, H), lambda i: (0, i, 0)),
            pl.BlockSpec((ROW_BLK, NWORK), lambda i: (i, 0)),
            pl.BlockSpec((ROW_BLK, H), lambda i: (i, 0)),
            pl.BlockSpec((H, H), lambda i: (0, 0)),
            pl.BlockSpec((1, H), lambda i: (0, 0)),
        ],
        out_specs=[
            pl.BlockSpec((ROW_BLK, H), lambda i: (i, 0)),
            pl.BlockSpec((8, H), lambda i: (0, 0)),
        ],
        out_shape=[
            jax.ShapeDtypeStruct((N, H), jnp.float32),
            jax.ShapeDtypeStruct((8, H), jnp.float32),
        ],
        scratch_shapes=[pltpu.VMEM((8, H), jnp.float32)],
    )(psum, pcnt, z, W_l, b_l2)


def _tc_post_b(pre, res, stats, gamma2, beta2):
    def body(pre_ref, res_ref, st_ref, g_ref, b_ref, out_ref):
        mu = st_ref[0:1] / N
        var = st_ref[1:2] / N - mu * mu
        scale = g_ref[...] * lax.rsqrt(var + 1e-5)
        y = (pre_ref[...] - mu) * scale + b_ref[...]
        out_ref[...] = jnp.where(y >= 0, y, 0.1 * y) + res_ref[...]

    return pl.pallas_call(
        body,
        grid=(NBLK,),
        in_specs=[
            pl.BlockSpec((ROW_BLK, H), lambda i: (i, 0)),
            pl.BlockSpec((ROW_BLK, H), lambda i: (i, 0)),
            pl.BlockSpec((8, H), lambda i: (0, 0)),
            pl.BlockSpec((1, H), lambda i: (0, 0)),
            pl.BlockSpec((1, H), lambda i: (0, 0)),
        ],
        out_specs=pl.BlockSpec((ROW_BLK, H), lambda i: (i, 0)),
        out_shape=jax.ShapeDtypeStruct((N, H), jnp.float32),
    )(pre, res, stats, gamma2, beta2)


def kernel(x, edge_index, W_l0, b_l0, W_r0, gamma0, beta0,
           W_l1, b_l1, W_r1, gamma1, beta1, skip_W, skip_b):
    src = edge_index[0]
    dst = edge_index[1]
    pad = E_PAD - E
    pad_iota = lax.iota(jnp.int32, pad)
    src_p = jnp.concatenate([src, (pad_iota * 37) % N])
    src_p = src_p.reshape(TOTAL_CHUNKS, CHUNK)
    junk = N + (pad_iota % (N_PAD - N))
    dst_p = jnp.concatenate([dst, junk]).reshape(TOTAL_CHUNKS, CHUNK)

    b_l0r = b_l0.reshape(1, H)
    b_l1r = b_l1.reshape(1, H)
    sbr = skip_b.reshape(1, H)
    g0r = gamma0.reshape(1, H)
    g1r = gamma1.reshape(1, H)
    be0r = beta0.reshape(1, H)
    be1r = beta1.reshape(1, H)

    z1, res1 = _tc_pre(x, W_r0, skip_W, sbr)
    (pc1,) = _sc_counts(dst_p)
    (ps1,) = _sc_aggregate(x, src_p, dst_p, with_cnt=True)
    ps1 = ps1.reshape(NCORES, N_PAD, H)
    pc1 = pc1.T
    pre1, st1 = _tc_post_a(ps1, pc1, z1, W_l0, b_l0r)
    h1 = _tc_post_b(pre1, res1, st1, g0r, be0r)

    z2, res2 = _tc_pre(h1, W_r1, skip_W, sbr)
    (ps2,) = _sc_aggregate(h1, src_p, dst_p, with_cnt=False)
    ps2 = ps2.reshape(NCORES, N_PAD, H)
    pre2, st2 = _tc_post_a(ps2, pc1, z2, W_l1, b_l1r)
    out = _tc_post_b(pre2, res2, st2, g1r, be1r)
    return out
